# baseline (speedup 1.0000x reference)
import functools

import jax
import jax.numpy as jnp
from jax import lax
from jax.experimental import pallas as pl
from jax.experimental.pallas import tpu as pltpu


def _se_pipeline_kernel(x_hbm, w1_ref, w2_ref, mlow_ref, blow_ref, bhigh_ref,
                        ones_ref, o_hbm, xbuf, obuf, in_sem, out_sem,
                        *, nb, inv_hw):
    base = pl.program_id(0) * nb

    def dma_in(slot, step):
        pltpu.make_async_copy(x_hbm.at[base + step], xbuf.at[slot],
                              in_sem.at[slot]).start()

    def wait_in(slot):
        pltpu.make_async_copy(xbuf.at[slot], xbuf.at[slot],
                              in_sem.at[slot]).wait()

    def dma_out(slot, step):
        pltpu.make_async_copy(obuf.at[slot], o_hbm.at[base + step],
                              out_sem.at[slot]).start()

    def wait_out(slot):
        pltpu.make_async_copy(obuf.at[slot], obuf.at[slot],
                              out_sem.at[slot]).wait()

    dma_in(0, 0)

    def body(step, _):
        cur = lax.rem(step, 2)
        nxt = lax.rem(step + 1, 2)

        @pl.when(step + 1 < nb)
        def _():
            dma_in(nxt, step + 1)

        wait_in(cur)

        @pl.when(step >= 2)
        def _():
            wait_out(cur)

        xb = xbuf[cur]
        mlow = mlow_ref[...]
        xl16 = (xb * mlow).astype(jnp.bfloat16)
        xa16 = xb.astype(jnp.bfloat16)
        rs_low = lax.dot_general(xl16, ones_ref[...], (((1,), (0,)), ((), ())),
                                 preferred_element_type=jnp.float32)
        rs_all = lax.dot_general(xa16, ones_ref[...], (((1,), (0,)), ((), ())),
                                 preferred_element_type=jnp.float32)
        rs_high = (rs_all - rs_low).astype(jnp.bfloat16)
        rs_low = rs_low.astype(jnp.bfloat16)
        pooled = (
            lax.dot_general(blow_ref[...], rs_low, (((0,), (0,)), ((), ())),
                            preferred_element_type=jnp.float32)
            + lax.dot_general(bhigh_ref[...], rs_high, (((0,), (0,)), ((), ())),
                              preferred_element_type=jnp.float32)
        ) * inv_hw
        h = jnp.maximum(
            lax.dot_general(w1_ref[...], pooled, (((1,), (0,)), ((), ())),
                            preferred_element_type=jnp.float32), 0.0)
        s = jax.nn.sigmoid(
            lax.dot_general(w2_ref[...], h, (((1,), (0,)), ((), ())),
                            preferred_element_type=jnp.float32))
        s16 = s.astype(jnp.bfloat16)
        s_low = lax.dot_general(blow_ref[...], s16, (((1,), (0,)), ((), ())),
                                preferred_element_type=jnp.float32)
        s_high = lax.dot_general(bhigh_ref[...], s16, (((1,), (0,)), ((), ())),
                                 preferred_element_type=jnp.float32)
        sexp = s_high + (s_low - s_high) * mlow
        obuf[cur] = xb * sexp
        dma_out(cur, step)
        return ()

    lax.fori_loop(0, nb, body, ())
    if nb >= 2:
        wait_out(lax.rem(nb - 2, 2))
    wait_out(lax.rem(nb - 1, 2))


def kernel(x, w1, w2):
    B, C, H, W = x.shape
    HW = H * W
    Cr = w1.shape[0]
    R = (C * HW) // 128

    xp = x.reshape(B, R, 128)
    row = lax.broadcasted_iota(jnp.int32, (R, 128), 0)
    lane = lax.broadcasted_iota(jnp.int32, (R, 128), 1)
    cl = (128 * row[:, :1]) // HW
    ch = (128 * row[:, :1] + 127) // HW
    mlow = ((128 * row + lane) // HW == cl).astype(jnp.float32)
    chan = lax.broadcasted_iota(jnp.int32, (R, C), 1)
    blow = (chan == cl).astype(jnp.bfloat16)
    bhigh = (chan == ch).astype(jnp.bfloat16)
    ones = jnp.ones((128, 1), jnp.bfloat16)

    n_cores = 2 if B % 2 == 0 else 1
    nb = B // n_cores

    body = functools.partial(_se_pipeline_kernel, nb=nb,
                             inv_hw=1.0 / float(HW))
    out = pl.pallas_call(
        body,
        out_shape=jax.ShapeDtypeStruct((B, R, 128), x.dtype),
        grid=(n_cores,),
        in_specs=[
            pl.BlockSpec(memory_space=pl.ANY),
            pl.BlockSpec((Cr, C), lambda i: (0, 0)),
            pl.BlockSpec((C, Cr), lambda i: (0, 0)),
            pl.BlockSpec((R, 128), lambda i: (0, 0)),
            pl.BlockSpec((R, C), lambda i: (0, 0)),
            pl.BlockSpec((R, C), lambda i: (0, 0)),
            pl.BlockSpec((128, 1), lambda i: (0, 0)),
        ],
        out_specs=pl.BlockSpec(memory_space=pl.ANY),
        scratch_shapes=[
            pltpu.VMEM((2, R, 128), x.dtype),
            pltpu.VMEM((2, R, 128), x.dtype),
            pltpu.SemaphoreType.DMA((2,)),
            pltpu.SemaphoreType.DMA((2,)),
        ],
        compiler_params=pltpu.CompilerParams(
            dimension_semantics=("parallel",),
        ),
    )(xp, w1, w2, mlow, blow, bhigh, ones)
    return out.reshape(B, C, H, W)


# speedup vs baseline: 3.9605x; 3.9605x over previous
import functools

import jax
import jax.numpy as jnp
from jax import lax
from jax.experimental import pallas as pl
from jax.experimental.pallas import tpu as pltpu


def _se_pipeline_kernel(x_hbm, w1_ref, w2_ref, o_hbm,
                        xbuf, obuf, in_sem, out_sem, *, n_steps, mb, inv_hw):
    base = pl.program_id(0) * n_steps * mb

    def dma_in(slot, step):
        pltpu.make_async_copy(x_hbm.at[pl.ds(base + step * mb, mb)],
                              xbuf.at[slot], in_sem.at[slot]).start()

    def wait_in(slot):
        pltpu.make_async_copy(xbuf.at[slot], xbuf.at[slot],
                              in_sem.at[slot]).wait()

    def dma_out(slot, step):
        pltpu.make_async_copy(obuf.at[slot], o_hbm.at[pl.ds(base + step * mb, mb)],
                              out_sem.at[slot]).start()

    def wait_out(slot):
        pltpu.make_async_copy(obuf.at[slot], obuf.at[slot],
                              out_sem.at[slot]).wait()

    dma_in(0, 0)

    def body(step, _):
        cur = lax.rem(step, 2)
        nxt = lax.rem(step + 1, 2)

        @pl.when(step + 1 < n_steps)
        def _():
            dma_in(nxt, step + 1)

        wait_in(cur)

        @pl.when(step >= 2)
        def _():
            wait_out(cur)

        for b in range(mb):
            xb = xbuf[cur, b]
            pooled = jnp.sum(xb, axis=1, keepdims=True) * inv_hw
            h = jnp.maximum(
                lax.dot_general(w1_ref[...], pooled,
                                (((1,), (0,)), ((), ())),
                                preferred_element_type=jnp.float32), 0.0)
            s = jax.nn.sigmoid(
                lax.dot_general(w2_ref[...], h, (((1,), (0,)), ((), ())),
                                preferred_element_type=jnp.float32))
            obuf[cur, b] = xb * s
        dma_out(cur, step)
        return ()

    lax.fori_loop(0, n_steps, body, ())
    if n_steps >= 2:
        wait_out(lax.rem(n_steps - 2, 2))
    wait_out(lax.rem(n_steps - 1, 2))


def kernel(x, w1, w2):
    B, C, H, W = x.shape
    HW = H * W
    Cr = w1.shape[0]

    x3 = x.reshape(B, C, HW)
    n_cores = 2 if B % 2 == 0 else 1
    mb = 4 if (B // n_cores) % 4 == 0 else 1
    n_steps = B // (n_cores * mb)

    body = functools.partial(_se_pipeline_kernel, n_steps=n_steps, mb=mb,
                             inv_hw=1.0 / float(HW))
    out3 = pl.pallas_call(
        body,
        out_shape=jax.ShapeDtypeStruct((B, C, HW), x.dtype),
        grid=(n_cores,),
        in_specs=[
            pl.BlockSpec(memory_space=pl.ANY),
            pl.BlockSpec((Cr, C), lambda i: (0, 0)),
            pl.BlockSpec((C, Cr), lambda i: (0, 0)),
        ],
        out_specs=pl.BlockSpec(memory_space=pl.ANY),
        scratch_shapes=[
            pltpu.VMEM((2, mb, C, HW), x.dtype),
            pltpu.VMEM((2, mb, C, HW), x.dtype),
            pltpu.SemaphoreType.DMA((2,)),
            pltpu.SemaphoreType.DMA((2,)),
        ],
        compiler_params=pltpu.CompilerParams(
            dimension_semantics=("parallel",),
            vmem_limit_bytes=64 * 1024 * 1024,
        ),
    )(x3, w1, w2)
    return out3.reshape(B, C, H, W)


# speedup vs baseline: 4.0709x; 1.0279x over previous
import functools

import jax
import jax.numpy as jnp
from jax import lax
from jax.experimental import pallas as pl
from jax.experimental.pallas import tpu as pltpu


def _se_pipeline_kernel(x_hbm, w1_ref, w2_ref, o_hbm,
                        xbuf, obuf, in_sem, out_sem, *, n_steps, mb, inv_hw):
    base = pl.program_id(0) * n_steps * mb

    def dma_in(slot, step):
        pltpu.make_async_copy(x_hbm.at[pl.ds(base + step * mb, mb)],
                              xbuf.at[slot], in_sem.at[slot]).start()

    def wait_in(slot):
        pltpu.make_async_copy(xbuf.at[slot], xbuf.at[slot],
                              in_sem.at[slot]).wait()

    def dma_out(slot, step):
        pltpu.make_async_copy(obuf.at[slot], o_hbm.at[pl.ds(base + step * mb, mb)],
                              out_sem.at[slot]).start()

    def wait_out(slot):
        pltpu.make_async_copy(obuf.at[slot], obuf.at[slot],
                              out_sem.at[slot]).wait()

    dma_in(0, 0)

    def body(step, _):
        cur = lax.rem(step, 3)
        nxt = lax.rem(step + 1, 3)

        @pl.when(step + 1 < n_steps)
        def _():
            dma_in(nxt, step + 1)

        wait_in(cur)

        @pl.when(step >= 3)
        def _():
            wait_out(cur)

        for b in range(mb):
            xb = xbuf[cur, b]
            pooled = jnp.sum(xb, axis=1, keepdims=True) * inv_hw
            h = jnp.maximum(
                lax.dot_general(w1_ref[...], pooled,
                                (((1,), (0,)), ((), ())),
                                preferred_element_type=jnp.float32), 0.0)
            s = jax.nn.sigmoid(
                lax.dot_general(w2_ref[...], h, (((1,), (0,)), ((), ())),
                                preferred_element_type=jnp.float32))
            obuf[cur, b] = xb * s
        dma_out(cur, step)
        return ()

    lax.fori_loop(0, n_steps, body, ())
    if n_steps >= 3:
        wait_out(lax.rem(n_steps - 3, 3))
    if n_steps >= 2:
        wait_out(lax.rem(n_steps - 2, 3))
    wait_out(lax.rem(n_steps - 1, 3))


def kernel(x, w1, w2):
    B, C, H, W = x.shape
    HW = H * W
    Cr = w1.shape[0]

    x3 = x.reshape(B, C, HW)
    n_cores = 1
    mb = 4 if (B // n_cores) % 4 == 0 else 1
    n_steps = B // (n_cores * mb)

    body = functools.partial(_se_pipeline_kernel, n_steps=n_steps, mb=mb,
                             inv_hw=1.0 / float(HW))
    out3 = pl.pallas_call(
        body,
        out_shape=jax.ShapeDtypeStruct((B, C, HW), x.dtype),
        grid=(n_cores,),
        in_specs=[
            pl.BlockSpec(memory_space=pl.ANY),
            pl.BlockSpec((Cr, C), lambda i: (0, 0)),
            pl.BlockSpec((C, Cr), lambda i: (0, 0)),
        ],
        out_specs=pl.BlockSpec(memory_space=pl.ANY),
        scratch_shapes=[
            pltpu.VMEM((3, mb, C, HW), x.dtype),
            pltpu.VMEM((3, mb, C, HW), x.dtype),
            pltpu.SemaphoreType.DMA((3,)),
            pltpu.SemaphoreType.DMA((3,)),
        ],
        compiler_params=pltpu.CompilerParams(
            dimension_semantics=("parallel",),
            vmem_limit_bytes=64 * 1024 * 1024,
        ),
    )(x3, w1, w2)
    return out3.reshape(B, C, H, W)
